# baseline (speedup 1.0000x reference)
import functools

import jax
import jax.numpy as jnp
from jax import lax
from jax.experimental import pallas as pl
from jax.experimental.pallas import tpu as pltpu
from jax.experimental.pallas import tpu_sc as plsc

N, C, T, V = 8, 128, 2048, 25
H = 8
NT = 512


def _vsum_kernel(xt_ref, s_ref):
    acc = xt_ref[0]
    for v in range(1, V):
        acc = acc + xt_ref[v]
    s_ref[...] = acc


def _stage_a(xt):
    Cb, Tb = 128, 512
    return pl.pallas_call(
        _vsum_kernel,
        grid=(N, C // Cb, T // Tb),
        in_specs=[pl.BlockSpec((None, V, Cb, Tb), lambda n, i, j: (n, 0, i, j))],
        out_specs=pl.BlockSpec((None, Cb, Tb), lambda n, i, j: (n, i, j)),
        out_shape=jax.ShapeDtypeStruct((N, C, T), jnp.float32),
    )(xt)


def _qk_kernel(s_ref, w_ref, b_ref, qk_ref):
    xn = s_ref[...] * jnp.float32(0.04)
    acc = lax.dot_general(w_ref[...], xn, (((1,), (0,)), ((), ())))
    qk_ref[...] = (acc + b_ref[...]).astype(jnp.bfloat16)


def _stage_b(s, W, b2):
    Tb = 512
    return pl.pallas_call(
        _qk_kernel,
        grid=(N, T // Tb),
        in_specs=[
            pl.BlockSpec((None, C, Tb), lambda n, j: (n, 0, j)),
            pl.BlockSpec((2 * C, C), lambda n, j: (0, 0)),
            pl.BlockSpec((2 * C, 1), lambda n, j: (0, 0)),
        ],
        out_specs=pl.BlockSpec((None, 2 * C, Tb), lambda n, j: (n, 0, j)),
        out_shape=jax.ShapeDtypeStruct((N, 2 * C, T), jnp.bfloat16),
    )(s, W, b2)


def _att_kernel(qt_ref, qk_ref, sc_ref):
    qk = qk_ref[...]
    qt = qt_ref[...]
    att3 = None
    for h in range(H):
        qh = lax.transpose(qt[32 * h:32 * h + 16, :], (1, 0))
        kh = lax.transpose(qk[32 * h + 16:32 * h + 32, :], (1, 0))
        prod = lax.dot_general(qh, kh, (((1,), (1,)), ((), ())),
                               preferred_element_type=jnp.float32)
        term = prod * jnp.float32(0.25)
        att3 = term if att3 is None else att3 + term
    acc = None
    for g in range(16):
        m = att3[:, g * 128:(g + 1) * 128] * jnp.float32(0.125)
        acc = m if acc is None else acc + m
    p = None
    for j2 in range(16):
        chunk = acc[:, 8 * j2:8 * j2 + 8]
        p = chunk if p is None else p + chunk
    t1 = p[:, :4] + p[:, 4:]
    t2 = t1[:, :2] + t1[:, 2:]
    res = t2[:, 0:1] + t2[:, 1:2]
    sc_ref[...] = res * jnp.float32(0.00048828125)


def _stage_c(qkb):
    Tb = 256
    return pl.pallas_call(
        _att_kernel,
        grid=(N, T // Tb),
        in_specs=[pl.BlockSpec((None, 2 * C, Tb), lambda n, j: (n, 0, j)),
                  pl.BlockSpec((None, 2 * C, T), lambda n, j: (n, 0, 0))],
        out_specs=pl.BlockSpec((None, Tb, 1), lambda n, j: (n, j, 0)),
        out_shape=jax.ShapeDtypeStruct((N, T, 1), jnp.float32),
    )(qkb, qkb)


def _sort_kernel(sc_ref, idx_ref, w_ref):
    v = sc_ref[...]
    lanes = lax.broadcasted_iota(jnp.int32, (N, T), 1)
    i = lanes
    k = 2
    while k <= T:
        j = k >> 1
        while j > 0:
            pv = jnp.where((lanes & j) == 0,
                           jnp.roll(v, -j, axis=1), jnp.roll(v, j, axis=1))
            pi = jnp.where((lanes & j) == 0,
                           jnp.roll(i, -j, axis=1), jnp.roll(i, j, axis=1))
            sg = (v > pv) | ((v == pv) & (i < pi))
            is_low = (lanes & j) == 0
            desc = (lanes & k) == 0
            take_mx = is_low == desc
            keep = take_mx == sg
            v = jnp.where(keep, v, pv)
            i = jnp.where(keep, i, pi)
            j >>= 1
        k <<= 1
    idx_ref[...] = i[:, :NT]
    sval = v[:, :NT]
    w_ref[...] = jnp.float32(1.0) / (jnp.exp(-sval) + jnp.float32(1.0))


def _stage_d(sc):
    return pl.pallas_call(
        _sort_kernel,
        grid=(1,),
        in_specs=[pl.BlockSpec((N, T), lambda _: (0, 0))],
        out_specs=[pl.BlockSpec((N, NT), lambda _: (0, 0)),
                   pl.BlockSpec((N, NT), lambda _: (0, 0))],
        out_shape=[jax.ShapeDtypeStruct((N, NT), jnp.int32),
                   jax.ShapeDtypeStruct((N, NT), jnp.float32)],
    )(sc)


def _gather_body(xr1, xidx, wexp, out, idx0_v, wexp_v, idxc_v, gat_v, sem):
    cid = lax.axis_index("c")
    sid = lax.axis_index("s")
    wid = sid * 2 + cid
    n = wid // 4
    c0 = (wid % 4) * 32
    pltpu.sync_copy(xidx.at[n], idx0_v)
    pltpu.sync_copy(wexp.at[n], wexp_v)

    def per_c(ci, carry):
        cc = c0 + ci
        base_v = jnp.full((16,), (n * 128 + cc) * (T * V), jnp.int32)

        def badd(i, c2):
            sl = pl.ds(i * 16, 16)
            idxc_v[sl] = idx0_v[sl] + base_v
            return c2

        lax.fori_loop(0, NT * V // 16, badd, 0)
        cp = pltpu.make_async_copy(xr1.at[idxc_v], gat_v, sem)
        cp.start()
        cp.wait()

        def bmul(i, c2):
            sl = pl.ds(i * 16, 16)
            gat_v[sl] = gat_v[sl] * wexp_v[sl]
            return c2

        lax.fori_loop(0, NT * V // 16, bmul, 0)
        pltpu.sync_copy(gat_v, out.at[n, cc])
        return carry

    lax.fori_loop(0, 32, per_c, 0)


def _stage_e(xr1, xidx, wexp):
    mesh = plsc.VectorSubcoreMesh(core_axis_name="c", subcore_axis_name="s")
    fn = functools.partial(
        pl.kernel,
        mesh=mesh,
        out_type=jax.ShapeDtypeStruct((N, C, NT * V), jnp.float32),
        scratch_types=[
            pltpu.VMEM((NT * V,), jnp.int32),
            pltpu.VMEM((NT * V,), jnp.float32),
            pltpu.VMEM((NT * V,), jnp.int32),
            pltpu.VMEM((NT * V,), jnp.float32),
            pltpu.SemaphoreType.DMA,
        ],
    )(_gather_body)
    return fn(xr1, xidx, wexp)


def kernel(x, W, b):
    xt = jnp.transpose(x, (0, 3, 1, 2))
    s = _stage_a(xt)
    qkb = _stage_b(s, W, b.reshape(2 * C, 1))
    sc = _stage_c(qkb).reshape(N, T)
    traw, wv = _stage_d(sc)
    idx0 = jnp.repeat(traw * V, V, axis=1) + jnp.tile(
        jnp.arange(V, dtype=jnp.int32), (N, NT))
    wexp = jnp.repeat(wv, V, axis=1)
    xr1 = x.reshape(-1)
    out = _stage_e(xr1, idx0, wexp)
    return out.reshape(N, C, NT, V)


# speedup vs baseline: 1.1108x; 1.1108x over previous
import functools

import jax
import jax.numpy as jnp
from jax import lax
from jax.experimental import pallas as pl
from jax.experimental.pallas import tpu as pltpu
from jax.experimental.pallas import tpu_sc as plsc

N, C, T, V = 8, 128, 2048, 25
H = 8
NT = 512


def _vsum_kernel(xt_ref, s_ref):
    acc = xt_ref[0]
    for v in range(1, V):
        acc = acc + xt_ref[v]
    s_ref[...] = acc


def _stage_a(xt):
    Cb, Tb = 128, 512
    return pl.pallas_call(
        _vsum_kernel,
        grid=(N, C // Cb, T // Tb),
        in_specs=[pl.BlockSpec((None, V, Cb, Tb), lambda n, i, j: (n, 0, i, j))],
        out_specs=pl.BlockSpec((None, Cb, Tb), lambda n, i, j: (n, i, j)),
        out_shape=jax.ShapeDtypeStruct((N, C, T), jnp.float32),
    )(xt)


def _qk_kernel(s_ref, w_ref, b_ref, qk_ref):
    xn = s_ref[...] * jnp.float32(0.04)
    acc = lax.dot_general(w_ref[...], xn, (((1,), (0,)), ((), ())))
    qk_ref[...] = (acc + b_ref[...]).astype(jnp.bfloat16)


def _stage_b(s, W, b2):
    Tb = 512
    return pl.pallas_call(
        _qk_kernel,
        grid=(N, T // Tb),
        in_specs=[
            pl.BlockSpec((None, C, Tb), lambda n, j: (n, 0, j)),
            pl.BlockSpec((2 * C, C), lambda n, j: (0, 0)),
            pl.BlockSpec((2 * C, 1), lambda n, j: (0, 0)),
        ],
        out_specs=pl.BlockSpec((None, 2 * C, Tb), lambda n, j: (n, 0, j)),
        out_shape=jax.ShapeDtypeStruct((N, 2 * C, T), jnp.bfloat16),
    )(s, W, b2)


def _att_kernel(qt_ref, qk_ref, sc_ref):
    qk = qk_ref[...]
    qt = qt_ref[...]
    att3 = None
    for h in range(H):
        qh = lax.transpose(qt[32 * h:32 * h + 16, :], (1, 0))
        kh = lax.transpose(qk[32 * h + 16:32 * h + 32, :], (1, 0))
        prod = lax.dot_general(qh, kh, (((1,), (1,)), ((), ())),
                               preferred_element_type=jnp.float32)
        term = prod * jnp.float32(0.25)
        att3 = term if att3 is None else att3 + term
    acc = None
    for g in range(16):
        m = att3[:, g * 128:(g + 1) * 128] * jnp.float32(0.125)
        acc = m if acc is None else acc + m
    p = None
    for j2 in range(16):
        chunk = acc[:, 8 * j2:8 * j2 + 8]
        p = chunk if p is None else p + chunk
    t1 = p[:, :4] + p[:, 4:]
    t2 = t1[:, :2] + t1[:, 2:]
    res = t2[:, 0:1] + t2[:, 1:2]
    sc_ref[...] = res * jnp.float32(0.00048828125)


def _stage_c(qkb):
    Tb = 256
    return pl.pallas_call(
        _att_kernel,
        grid=(N, T // Tb),
        in_specs=[pl.BlockSpec((None, 2 * C, Tb), lambda n, j: (n, 0, j)),
                  pl.BlockSpec((None, 2 * C, T), lambda n, j: (n, 0, 0))],
        out_specs=pl.BlockSpec((None, Tb, 1), lambda n, j: (n, j, 0)),
        out_shape=jax.ShapeDtypeStruct((N, T, 1), jnp.float32),
    )(qkb, qkb)


def _sort_kernel(sc_ref, idx_ref, w_ref):
    v = sc_ref[...]
    lanes = lax.broadcasted_iota(jnp.int32, (N, T), 1)
    i = lanes
    k = 2
    while k <= T:
        j = k >> 1
        while j > 0:
            pv = jnp.where((lanes & j) == 0,
                           jnp.roll(v, -j, axis=1), jnp.roll(v, j, axis=1))
            pi = jnp.where((lanes & j) == 0,
                           jnp.roll(i, -j, axis=1), jnp.roll(i, j, axis=1))
            sg = (v > pv) | ((v == pv) & (i < pi))
            is_low = (lanes & j) == 0
            desc = (lanes & k) == 0
            take_mx = is_low == desc
            keep = take_mx == sg
            v = jnp.where(keep, v, pv)
            i = jnp.where(keep, i, pi)
            j >>= 1
        k <<= 1
    idx_ref[...] = i[:, :NT]
    sval = v[:, :NT]
    w_ref[...] = jnp.float32(1.0) / (jnp.exp(-sval) + jnp.float32(1.0))


def _stage_d(sc):
    return pl.pallas_call(
        _sort_kernel,
        grid=(1,),
        in_specs=[pl.BlockSpec((N, T), lambda _: (0, 0))],
        out_specs=[pl.BlockSpec((N, NT), lambda _: (0, 0)),
                   pl.BlockSpec((N, NT), lambda _: (0, 0))],
        out_shape=[jax.ShapeDtypeStruct((N, NT), jnp.int32),
                   jax.ShapeDtypeStruct((N, NT), jnp.float32)],
    )(sc)


def _gather_body(xr1, xidx, wexp, out, idx0_v, wexp_v, idxc_v, gat_v, idxb_v, gatb_v, sem, semb):
    cid = lax.axis_index("c")
    sid = lax.axis_index("s")
    wid = sid * 2 + cid
    n = wid // 4
    c0 = (wid % 4) * 32
    pltpu.sync_copy(xidx.at[n], idx0_v)
    pltpu.sync_copy(wexp.at[n], wexp_v)

    def build_start(idx_v, g_v, cc, sem_):
        base_v = jnp.full((16,), (n * 128 + cc) * (T * V), jnp.int32)

        def badd(i, c2):
            for u in range(8):
                sl = pl.ds(i * 128 + u * 16, 16)
                idx_v[sl] = idx0_v[sl] + base_v
            return c2

        lax.fori_loop(0, NT * V // 128, badd, 0)
        pltpu.make_async_copy(xr1.at[idx_v], g_v, sem_).start()

    def finish(idx_v, g_v, cc, sem_):
        pltpu.make_async_copy(xr1.at[idx_v], g_v, sem_).wait()

        def bmul(i, c2):
            for u in range(8):
                sl = pl.ds(i * 128 + u * 16, 16)
                g_v[sl] = g_v[sl] * wexp_v[sl]
            return c2

        lax.fori_loop(0, NT * V // 128, bmul, 0)
        pltpu.sync_copy(g_v, out.at[n, cc])

    def per_pair(i, carry):
        ca = c0 + 2 * i
        cb = ca + 1
        build_start(idxc_v, gat_v, ca, sem)
        build_start(idxb_v, gatb_v, cb, semb)
        finish(idxc_v, gat_v, ca, sem)
        finish(idxb_v, gatb_v, cb, semb)
        return carry

    lax.fori_loop(0, 16, per_pair, 0)


def _stage_e(xr1, xidx, wexp):
    mesh = plsc.VectorSubcoreMesh(core_axis_name="c", subcore_axis_name="s")
    fn = functools.partial(
        pl.kernel,
        mesh=mesh,
        out_type=jax.ShapeDtypeStruct((N, C, NT * V), jnp.float32),
        scratch_types=[
            pltpu.VMEM((NT * V,), jnp.int32),
            pltpu.VMEM((NT * V,), jnp.float32),
            pltpu.VMEM((NT * V,), jnp.int32),
            pltpu.VMEM((NT * V,), jnp.float32),
            pltpu.VMEM((NT * V,), jnp.int32),
            pltpu.VMEM((NT * V,), jnp.float32),
            pltpu.SemaphoreType.DMA,
            pltpu.SemaphoreType.DMA,
        ],
    )(_gather_body)
    return fn(xr1, xidx, wexp)


def kernel(x, W, b):
    xt = jnp.transpose(x, (0, 3, 1, 2))
    s = _stage_a(xt)
    qkb = _stage_b(s, W, b.reshape(2 * C, 1))
    sc = _stage_c(qkb).reshape(N, T)
    traw, wv = _stage_d(sc)
    idx0 = jnp.repeat(traw * V, V, axis=1) + jnp.tile(
        jnp.arange(V, dtype=jnp.int32), (N, NT))
    wexp = jnp.repeat(wv, V, axis=1)
    xr1 = x.reshape(-1)
    out = _stage_e(xr1, idx0, wexp)
    return out.reshape(N, C, NT, V)
